# baseline (device time: 112670 ns/iter reference)
import jax
import jax.numpy as jnp
from jax import lax
from jax.experimental import pallas as pl
from jax.experimental.pallas import tpu as pltpu

N_DEV = 8
SQ = 2048
SKV_LOCAL = 2048
HQ = 8
DH = 128
DM = HQ * DH
QBLK = 256
NCHUNK = SQ // QBLK
SCALE = 0.08838834764831843

CHILDREN = {0: (1, 3, 4), 1: (2, 5), 2: (6,), 3: (7,), 4: (), 5: (), 6: (), 7: ()}
PARENT = {1: 0, 2: 1, 3: 0, 4: 0, 5: 1, 6: 2, 7: 3}


def kernel(x, Wq, K_ext, V_ext, Wo):
    x2 = x[0]
    k2 = K_ext[0].reshape(SKV_LOCAL, DM)
    v2 = V_ext[0].reshape(SKV_LOCAL, DM)

    def body(x_ref, wq_ref, k_ref, v_ref, wo_ref, out_ref,
             ctx_ref, kbf_ref, vbf_ref, wqbf_ref, wobf_ref,
             send_sems, recv_sems):
        my = lax.axis_index("i")

        wobf_ref[...] = wo_ref[...].astype(jnp.bfloat16)

        def chunk_copy(b, target, d):
            return pltpu.make_async_remote_copy(
                src_ref=ctx_ref.at[pl.ds(b * QBLK, QBLK)],
                dst_ref=ctx_ref.at[pl.ds(b * QBLK, QBLK)],
                send_sem=send_sems.at[d, b],
                recv_sem=recv_sems.at[b],
                device_id=(target,), device_id_type=pl.DeviceIdType.MESH,
            )

        def project(b):
            out_ref[b * QBLK:(b + 1) * QBLK, :] = jnp.dot(
                ctx_ref[b * QBLK:(b + 1) * QBLK, :], wobf_ref[...],
                preferred_element_type=jnp.float32,
            ).astype(jnp.bfloat16)

        def barrier(peers):
            bsem = pltpu.get_barrier_semaphore()
            for pr in peers:
                pl.semaphore_signal(
                    bsem, inc=1,
                    device_id=(pr,), device_id_type=pl.DeviceIdType.MESH,
                )
            pl.semaphore_wait(bsem, len(peers))

        @pl.when(my == 0)
        def _():
            barrier(CHILDREN[0])
            kbf_ref[...] = k_ref[...].astype(jnp.bfloat16)
            vbf_ref[...] = v_ref[...].astype(jnp.bfloat16)
            wqbf_ref[...] = wq_ref[...].astype(jnp.bfloat16)
            for b in range(NCHUNK):
                kmax = (b + 1) * QBLK
                xb = x_ref[b * QBLK:(b + 1) * QBLK, :].astype(jnp.bfloat16)
                qb = jnp.dot(
                    xb, wqbf_ref[...], preferred_element_type=jnp.float32
                ).astype(jnp.bfloat16)
                rows = b * QBLK + lax.broadcasted_iota(
                    jnp.int32, (QBLK, kmax), 0)
                cols = lax.broadcasted_iota(jnp.int32, (QBLK, kmax), 1)
                mask = (cols // 64) <= (rows // 64)
                for h in range(HQ):
                    kh = kbf_ref[:kmax, h * DH:(h + 1) * DH]
                    vh = vbf_ref[:kmax, h * DH:(h + 1) * DH]
                    qh = qb[:, h * DH:(h + 1) * DH]
                    s = lax.dot_general(
                        qh, kh, (((1,), (1,)), ((), ())),
                        preferred_element_type=jnp.float32,
                    ) * SCALE
                    s = jnp.where(mask, s, -1e9)
                    p32 = jnp.exp(s)
                    l = jnp.sum(p32, axis=-1, keepdims=True)
                    cun = jnp.dot(
                        p32.astype(jnp.bfloat16), vh,
                        preferred_element_type=jnp.float32,
                    )
                    ctx_ref[b * QBLK:(b + 1) * QBLK, h * DH:(h + 1) * DH] = (
                        (cun * (1.0 / l)).astype(jnp.bfloat16)
                    )
                for d, tgt in enumerate(CHILDREN[0]):
                    chunk_copy(b, tgt, d).start()
            for b in range(NCHUNK):
                project(b)
            for b in range(NCHUNK):
                for d, tgt in enumerate(CHILDREN[0]):
                    chunk_copy(b, tgt, d).wait_send()

        for pos in range(1, N_DEV):
            children = CHILDREN[pos]

            @pl.when(my == pos)
            def _(pos=pos, children=children):
                barrier((PARENT[pos],) + children)
                for b in range(NCHUNK):
                    chunk_copy(b, PARENT[pos], 0).wait_recv()
                    for d, tgt in enumerate(children):
                        chunk_copy(b, tgt, d).start()
                    project(b)
                for b in range(NCHUNK):
                    for d, tgt in enumerate(children):
                        chunk_copy(b, tgt, d).wait_send()

    out = pl.pallas_call(
        body,
        out_shape=jax.ShapeDtypeStruct((SQ, DM), jnp.bfloat16),
        in_specs=[pl.BlockSpec(memory_space=pltpu.VMEM)] * 5,
        out_specs=pl.BlockSpec(memory_space=pltpu.VMEM),
        scratch_shapes=[
            pltpu.VMEM((SQ, DM), jnp.bfloat16),
            pltpu.VMEM((SKV_LOCAL, DM), jnp.bfloat16),
            pltpu.VMEM((SKV_LOCAL, DM), jnp.bfloat16),
            pltpu.VMEM((DM, DM), jnp.bfloat16),
            pltpu.VMEM((DM, DM), jnp.bfloat16),
            pltpu.SemaphoreType.DMA((3, NCHUNK)),
            pltpu.SemaphoreType.DMA((NCHUNK,)),
        ],
        compiler_params=pltpu.CompilerParams(
            collective_id=0, vmem_limit_bytes=64 * 1024 * 1024
        ),
    )(x2, Wq, k2, v2, Wo)
    return out.reshape(1, SQ, DM)
